# Initial kernel scaffold; baseline (speedup 1.0000x reference)
import jax
import jax.numpy as jnp
from jax.experimental import pallas as pl


def kernel(input, table, W, b):
    raise NotImplementedError("write your pallas kernel here")


# speedup vs baseline: 95.0349x; 95.0349x over previous
import functools

import jax
import jax.numpy as jnp
from jax import lax
from jax.experimental import pallas as pl
from jax.experimental.pallas import tpu as pltpu
from jax.experimental.pallas import tpu_sc as plsc

B = 16384
L = 200
N = B * L
DIM = 128
VOCAB = 10

NC = 2
NS = 16
NW = NC * NS
PER_W = N // NW
CH = 25600
NCH = PER_W // CH
LANES = 16


def _sc_body(idx_hbm, tabT_hbm, w_hbm, b_hbm, out_hbm,
             idx_v, out_v, tabT_v, w_v, b_v):
    pltpu.sync_copy(tabT_hbm, tabT_v)
    pltpu.sync_copy(w_hbm, w_v)
    pltpu.sync_copy(b_hbm, b_v)

    scores = b_v[...]
    for c in range(DIM):
        scores = scores + tabT_v[c, :] * w_v[c, :]
    sv = [jnp.broadcast_to(scores[v], (LANES,)) for v in range(VOCAB)]

    wid = lax.axis_index("s") * NC + lax.axis_index("c")
    base = wid * PER_W

    def chunk_body(ci, carry):
        off = base + ci * CH
        pltpu.sync_copy(idx_hbm.at[pl.ds(off, CH)], idx_v)

        def inner(i, c2):
            iv = idx_v[pl.ds(i * LANES, LANES)]
            b0 = (iv & 1) != 0
            b1 = (iv & 2) != 0
            b2 = (iv & 4) != 0
            b3 = (iv & 8) != 0
            t01 = jnp.where(b0, sv[1], sv[0])
            t23 = jnp.where(b0, sv[3], sv[2])
            t45 = jnp.where(b0, sv[5], sv[4])
            t67 = jnp.where(b0, sv[7], sv[6])
            t89 = jnp.where(b0, sv[9], sv[8])
            u0 = jnp.where(b1, t23, t01)
            u1 = jnp.where(b1, t67, t45)
            v0 = jnp.where(b2, u1, u0)
            out_v[pl.ds(i * LANES, LANES)] = jnp.where(b3, t89, v0)
            return c2

        lax.fori_loop(0, CH // LANES, inner, 0)
        pltpu.sync_copy(out_v, out_hbm.at[pl.ds(off, CH)])
        return carry

    lax.fori_loop(0, NCH, chunk_body, 0)


_sc_call = functools.partial(
    pl.kernel,
    out_type=jax.ShapeDtypeStruct((N,), jnp.float32),
    mesh=plsc.VectorSubcoreMesh(core_axis_name="c", subcore_axis_name="s"),
    scratch_types=[
        pltpu.VMEM((CH,), jnp.int32),
        pltpu.VMEM((CH,), jnp.float32),
        pltpu.VMEM((DIM, LANES), jnp.float32),
        pltpu.VMEM((DIM, LANES), jnp.float32),
        pltpu.VMEM((LANES,), jnp.float32),
    ],
)(_sc_body)


def kernel(input, table, W, b):
    idx = input.reshape(N).astype(jnp.int32)
    tabT = jnp.pad(table.T, ((0, 0), (0, LANES - VOCAB)))
    w16 = jnp.broadcast_to(W.reshape(DIM, 1), (DIM, LANES))
    b16 = jnp.broadcast_to(b, (LANES,))
    out_flat = _sc_call(idx, tabT, w16, b16)
    return out_flat.reshape(B, L, 1)


# speedup vs baseline: 95.2814x; 1.0026x over previous
import functools

import jax
import jax.numpy as jnp
from jax import lax
from jax.experimental import pallas as pl
from jax.experimental.pallas import tpu as pltpu
from jax.experimental.pallas import tpu_sc as plsc

B = 16384
L = 200
N = B * L
DIM = 128
VOCAB = 10

NC = 2
NS = 16
NW = NC * NS
PER_W = N // NW
CH = 25600
NCH = PER_W // CH
LANES = 16
UNROLL = 8


def _sc_body(idx_hbm, tabT_hbm, w_hbm, b_hbm, out_hbm,
             idx_v, out_v, tabT_v, w_v, b_v):
    pltpu.sync_copy(tabT_hbm, tabT_v)
    pltpu.sync_copy(w_hbm, w_v)
    pltpu.sync_copy(b_hbm, b_v)

    scores = b_v[...]
    for c in range(DIM):
        scores = scores + tabT_v[c, :] * w_v[c, :]
    sv = [jnp.broadcast_to(scores[v], (LANES,)) for v in range(VOCAB)]

    wid = lax.axis_index("s") * NC + lax.axis_index("c")
    base = wid * PER_W

    def chunk_body(ci, carry):
        off = base + ci * CH
        pltpu.sync_copy(idx_hbm.at[pl.ds(off, CH)], idx_v)

        def inner(i, c2):
            for j in range(UNROLL):
                o = i * (UNROLL * LANES) + j * LANES
                iv = idx_v[pl.ds(o, LANES)]
                b0 = (iv & 1) != 0
                b1 = (iv & 2) != 0
                b2 = (iv & 4) != 0
                b3 = (iv & 8) != 0
                t01 = jnp.where(b0, sv[1], sv[0])
                t23 = jnp.where(b0, sv[3], sv[2])
                t45 = jnp.where(b0, sv[5], sv[4])
                t67 = jnp.where(b0, sv[7], sv[6])
                t89 = jnp.where(b0, sv[9], sv[8])
                u0 = jnp.where(b1, t23, t01)
                u1 = jnp.where(b1, t67, t45)
                v0 = jnp.where(b2, u1, u0)
                out_v[pl.ds(o, LANES)] = jnp.where(b3, t89, v0)
            return c2

        lax.fori_loop(0, CH // (UNROLL * LANES), inner, 0)
        pltpu.sync_copy(out_v, out_hbm.at[pl.ds(off, CH)])
        return carry

    lax.fori_loop(0, NCH, chunk_body, 0)


_sc_call = functools.partial(
    pl.kernel,
    out_type=jax.ShapeDtypeStruct((N,), jnp.float32),
    mesh=plsc.VectorSubcoreMesh(core_axis_name="c", subcore_axis_name="s"),
    scratch_types=[
        pltpu.VMEM((CH,), jnp.int32),
        pltpu.VMEM((CH,), jnp.float32),
        pltpu.VMEM((DIM, LANES), jnp.float32),
        pltpu.VMEM((DIM, LANES), jnp.float32),
        pltpu.VMEM((LANES,), jnp.float32),
    ],
)(_sc_body)


def kernel(input, table, W, b):
    idx = input.reshape(N).astype(jnp.int32)
    tabT = jnp.pad(table.T, ((0, 0), (0, LANES - VOCAB)))
    w16 = jnp.broadcast_to(W.reshape(DIM, 1), (DIM, LANES))
    b16 = jnp.broadcast_to(b, (LANES,))
    out_flat = _sc_call(idx, tabT, w16, b16)
    return out_flat.reshape(B, L, 1)


# speedup vs baseline: 136.1965x; 1.4294x over previous
import functools

import jax
import jax.numpy as jnp
from jax import lax
from jax.experimental import pallas as pl
from jax.experimental.pallas import tpu as pltpu
from jax.experimental.pallas import tpu_sc as plsc

B = 16384
L = 200
DIM = 128
VOCAB = 10

NC = 2
NS = 16
NW = NC * NS
ROWS_W = B // NW
RCH = 128
NRCH = ROWS_W // RCH
LANES = 16
OFFS = tuple(range(0, L - LANES + 1, LANES)) + (L - LANES,)


def _sc_body(idx_hbm, tabT_hbm, w_hbm, b_hbm, out_hbm,
             idx_v, out_v, tabT_v, w_v, b_v):
    pltpu.sync_copy(tabT_hbm, tabT_v)
    pltpu.sync_copy(w_hbm, w_v)
    pltpu.sync_copy(b_hbm, b_v)

    scores = b_v[...]
    for c in range(DIM):
        scores = scores + tabT_v[c, :] * w_v[c, :]
    sv = [jnp.broadcast_to(scores[v], (LANES,)) for v in range(VOCAB)]

    wid = lax.axis_index("s") * NC + lax.axis_index("c")
    row0 = wid * ROWS_W

    def chunk_body(ci, carry):
        r0 = row0 + ci * RCH
        pltpu.sync_copy(idx_hbm.at[pl.ds(r0, RCH), :], idx_v)

        def row_body(r, c2):
            for off in OFFS:
                iv = idx_v[r, pl.ds(off, LANES)]
                b0 = (iv & 1) != 0
                b1 = (iv & 2) != 0
                b2 = (iv & 4) != 0
                b3 = (iv & 8) != 0
                t01 = jnp.where(b0, sv[1], sv[0])
                t23 = jnp.where(b0, sv[3], sv[2])
                t45 = jnp.where(b0, sv[5], sv[4])
                t67 = jnp.where(b0, sv[7], sv[6])
                t89 = jnp.where(b0, sv[9], sv[8])
                u0 = jnp.where(b1, t23, t01)
                u1 = jnp.where(b1, t67, t45)
                v0 = jnp.where(b2, u1, u0)
                out_v[r, pl.ds(off, LANES)] = jnp.where(b3, t89, v0)
            return c2

        lax.fori_loop(0, RCH, row_body, 0)
        pltpu.sync_copy(out_v, out_hbm.at[pl.ds(r0, RCH), :])
        return carry

    lax.fori_loop(0, NRCH, chunk_body, 0)


_sc_call = functools.partial(
    pl.kernel,
    out_type=jax.ShapeDtypeStruct((B, L), jnp.float32),
    mesh=plsc.VectorSubcoreMesh(core_axis_name="c", subcore_axis_name="s"),
    scratch_types=[
        pltpu.VMEM((RCH, L), jnp.int32),
        pltpu.VMEM((RCH, L), jnp.float32),
        pltpu.VMEM((DIM, LANES), jnp.float32),
        pltpu.VMEM((DIM, LANES), jnp.float32),
        pltpu.VMEM((LANES,), jnp.float32),
    ],
)(_sc_body)


def kernel(input, table, W, b):
    idx = input.astype(jnp.int32)
    tabT = jnp.pad(table.T, ((0, 0), (0, LANES - VOCAB)))
    w16 = jnp.broadcast_to(W.reshape(DIM, 1), (DIM, LANES))
    b16 = jnp.broadcast_to(b, (LANES,))
    out = _sc_call(idx, tabT, w16, b16)
    return out.reshape(B, L, 1)


# speedup vs baseline: 136.3468x; 1.0011x over previous
import functools

import jax
import jax.numpy as jnp
from jax import lax
from jax.experimental import pallas as pl
from jax.experimental.pallas import tpu as pltpu
from jax.experimental.pallas import tpu_sc as plsc

B = 16384
L = 200
DIM = 128
VOCAB = 10

NC = 2
NS = 16
NW = NC * NS
ROWS_W = B // NW
RCH = 128
NRCH = ROWS_W // RCH
LANES = 16
OFFS = tuple(range(0, L - LANES + 1, LANES)) + (L - LANES,)


def _sc_body(idx_hbm, tabT_hbm, w_hbm, b_hbm, out_hbm,
             idx_v, out_v, tabT_v, w_v, b_v):
    pltpu.sync_copy(tabT_hbm, tabT_v)
    pltpu.sync_copy(w_hbm, w_v)
    pltpu.sync_copy(b_hbm, b_v)

    scores = b_v[...]
    for c in range(DIM):
        scores = scores + tabT_v[c, :] * w_v[c, :]
    sv = [jnp.broadcast_to(scores[v], (LANES,)) for v in range(VOCAB)]

    wid = lax.axis_index("s") * NC + lax.axis_index("c")
    row0 = wid * ROWS_W

    def chunk_body(ci, carry):
        r0 = row0 + ci * RCH
        pltpu.sync_copy(idx_hbm.at[pl.ds(r0, RCH), :], idx_v)

        def row_body(r, c2):
            for off in OFFS:
                iv = idx_v[r, pl.ds(off, LANES)]
                b0 = (iv & 1) != 0
                b1 = (iv & 2) != 0
                b2 = (iv & 4) != 0
                b3 = (iv & 8) != 0
                t01 = jnp.where(b0, sv[1], sv[0])
                t23 = jnp.where(b0, sv[3], sv[2])
                t45 = jnp.where(b0, sv[5], sv[4])
                t67 = jnp.where(b0, sv[7], sv[6])
                t89 = jnp.where(b0, sv[9], sv[8])
                u0 = jnp.where(b1, t23, t01)
                u1 = jnp.where(b1, t67, t45)
                v0 = jnp.where(b2, u1, u0)
                out_v[r, pl.ds(off, LANES)] = jnp.where(b3, t89, v0)
            return c2

        lax.fori_loop(0, RCH, row_body, 0)
        pltpu.sync_copy(out_v, out_hbm.at[pl.ds(r0, RCH), :])
        return carry

    lax.fori_loop(0, NRCH, chunk_body, 0)


_sc_call = functools.partial(
    pl.kernel,
    out_type=jax.ShapeDtypeStruct((B, L), jnp.float32),
    compiler_params=pltpu.CompilerParams(use_tc_tiling_on_sc=True),
    mesh=plsc.VectorSubcoreMesh(core_axis_name="c", subcore_axis_name="s"),
    scratch_types=[
        pltpu.VMEM((RCH, L), jnp.int32),
        pltpu.VMEM((RCH, L), jnp.float32),
        pltpu.VMEM((DIM, LANES), jnp.float32),
        pltpu.VMEM((DIM, LANES), jnp.float32),
        pltpu.VMEM((LANES,), jnp.float32),
    ],
)(_sc_body)


def kernel(input, table, W, b):
    idx = input.astype(jnp.int32)
    tabT = jnp.pad(table.T, ((0, 0), (0, LANES - VOCAB)))
    w16 = jnp.broadcast_to(W.reshape(DIM, 1), (DIM, LANES))
    b16 = jnp.broadcast_to(b, (LANES,))
    out = _sc_call(idx, tabT, w16, b16)
    return out.reshape(B, L, 1)


# speedup vs baseline: 149.8344x; 1.0989x over previous
import functools

import jax
import jax.numpy as jnp
from jax import lax
from jax.experimental import pallas as pl
from jax.experimental.pallas import tpu as pltpu
from jax.experimental.pallas import tpu_sc as plsc

B = 16384
L = 200
DIM = 128
VOCAB = 10

NC = 2
NS = 16
NW = NC * NS
ROWS_W = B // NW
RCH = 64
NRCH = ROWS_W // RCH
LANES = 16
RUN = 2
OFFS = tuple(range(0, L - LANES + 1, LANES)) + (L - LANES,)


def _lookup16(iv, sv):
    b0 = (iv & 1) != 0
    b1 = (iv & 2) != 0
    b2 = (iv & 4) != 0
    b3 = (iv & 8) != 0
    t01 = jnp.where(b0, sv[1], sv[0])
    t23 = jnp.where(b0, sv[3], sv[2])
    t45 = jnp.where(b0, sv[5], sv[4])
    t67 = jnp.where(b0, sv[7], sv[6])
    t89 = jnp.where(b0, sv[9], sv[8])
    u0 = jnp.where(b1, t23, t01)
    u1 = jnp.where(b1, t67, t45)
    v0 = jnp.where(b2, u1, u0)
    return jnp.where(b3, t89, v0)


def _sc_body(idx_hbm, tabT_hbm, w_hbm, b_hbm, out_hbm,
             idx0_v, idx1_v, out0_v, out1_v, tabT_v, w_v, b_v,
             isem0, isem1, osem0, osem1):
    pltpu.sync_copy(tabT_hbm, tabT_v)
    pltpu.sync_copy(w_hbm, w_v)
    pltpu.sync_copy(b_hbm, b_v)

    scores = b_v[...]
    for c in range(DIM):
        scores = scores + tabT_v[c, :] * w_v[c, :]
    sv = [jnp.broadcast_to(scores[v], (LANES,)) for v in range(VOCAB)]

    wid = lax.axis_index("s") * NC + lax.axis_index("c")
    row0 = wid * ROWS_W

    ibuf = (idx0_v, idx1_v)
    obuf = (out0_v, out1_v)
    isem = (isem0, isem1)
    osem = (osem0, osem1)

    def start_in(ci, s):
        r0 = row0 + ci * RCH
        return pltpu.async_copy(idx_hbm.at[pl.ds(r0, RCH), :], ibuf[s], isem[s])

    def start_out(ci, s):
        r0 = row0 + ci * RCH
        return pltpu.async_copy(obuf[s], out_hbm.at[pl.ds(r0, RCH), :], osem[s])

    in_cp = {0: start_in(0, 0)}
    out_cp = {}
    for ci in range(NRCH):
        s = ci & 1
        in_cp[ci].wait()
        if ci + 1 < NRCH:
            in_cp[ci + 1] = start_in(ci + 1, 1 - s)
        if ci >= 2:
            out_cp[ci - 2].wait()
        idx_v, out_v = ibuf[s], obuf[s]

        def run_body(r2, c2, idx_v=idx_v, out_v=out_v):
            for rr in range(RUN):
                r = r2 * RUN + rr
                for off in OFFS:
                    iv = idx_v[r, pl.ds(off, LANES)]
                    out_v[r, pl.ds(off, LANES)] = _lookup16(iv, sv)
            return c2

        lax.fori_loop(0, RCH // RUN, run_body, 0)
        out_cp[ci] = start_out(ci, s)
    out_cp[NRCH - 2].wait()
    out_cp[NRCH - 1].wait()


_sc_call = functools.partial(
    pl.kernel,
    out_type=jax.ShapeDtypeStruct((B, L), jnp.float32),
    mesh=plsc.VectorSubcoreMesh(core_axis_name="c", subcore_axis_name="s"),
    scratch_types=[
        pltpu.VMEM((RCH, L), jnp.int32),
        pltpu.VMEM((RCH, L), jnp.int32),
        pltpu.VMEM((RCH, L), jnp.float32),
        pltpu.VMEM((RCH, L), jnp.float32),
        pltpu.VMEM((DIM, LANES), jnp.float32),
        pltpu.VMEM((DIM, LANES), jnp.float32),
        pltpu.VMEM((LANES,), jnp.float32),
        pltpu.SemaphoreType.DMA,
        pltpu.SemaphoreType.DMA,
        pltpu.SemaphoreType.DMA,
        pltpu.SemaphoreType.DMA,
    ],
)(_sc_body)


def kernel(input, table, W, b):
    idx = input.astype(jnp.int32)
    tabT = jnp.pad(table.T, ((0, 0), (0, LANES - VOCAB)))
    w16 = jnp.broadcast_to(W.reshape(DIM, 1), (DIM, LANES))
    b16 = jnp.broadcast_to(b, (LANES,))
    out = _sc_call(idx, tabT, w16, b16)
    return out.reshape(B, L, 1)
